# baseline (speedup 1.0000x reference)
import jax
import jax.numpy as jnp
from jax.experimental import pallas as pl
from jax.experimental.pallas import tpu as pltpu

B = 8
P = 2048
Q = 2048
DPAD = 8
QT = 512
NQT = Q // QT


def _chamfer_body(x_ref, yt_ref, out_ref, minx_ref, acc_ref):
    b = pl.program_id(0)
    qt = pl.program_id(1)

    x = x_ref[0]
    yt = yt_ref[0]

    xy = jax.lax.dot_general(
        x, yt, (((1,), (0,)), ((), ())), preferred_element_type=jnp.float32
    )
    x2 = jnp.sum(x * x, axis=1, keepdims=True)
    y2 = jnp.sum(yt * yt, axis=0, keepdims=True)
    d = jnp.maximum(x2 + y2 - 2.0 * xy, 0.0)

    tile_min_q = jnp.min(d, axis=1, keepdims=True)

    @pl.when(qt == 0)
    def _():
        minx_ref[...] = tile_min_q

    @pl.when(qt != 0)
    def _():
        minx_ref[...] = jnp.minimum(minx_ref[...], tile_min_q)

    miny = jnp.min(d, axis=0)
    s = jnp.sum(miny)

    @pl.when(jnp.logical_and(b == 0, qt == 0))
    def _():
        acc_ref[0, 0] = 0.0

    acc_ref[0, 0] += s

    @pl.when(qt == NQT - 1)
    def _():
        acc_ref[0, 0] += jnp.sum(minx_ref[...])

    @pl.when(jnp.logical_and(b == B - 1, qt == NQT - 1))
    def _():
        out_ref[0, 0] = acc_ref[0, 0] / (float(P) * float(B))


def kernel(x_hat, pos):
    xp = jnp.pad(x_hat, ((0, 0), (0, 0), (0, DPAD - 3)))
    ytp = jnp.pad(pos, ((0, 0), (0, 0), (0, DPAD - 3)))
    ytp = jnp.transpose(ytp, (0, 2, 1))

    out = pl.pallas_call(
        _chamfer_body,
        grid=(B, NQT),
        in_specs=[
            pl.BlockSpec((1, P, DPAD), lambda b, q: (b, 0, 0)),
            pl.BlockSpec((1, DPAD, QT), lambda b, q: (b, 0, q)),
        ],
        out_specs=pl.BlockSpec(
            (1, 1), lambda b, q: (0, 0), memory_space=pltpu.SMEM
        ),
        out_shape=jax.ShapeDtypeStruct((1, 1), jnp.float32),
        scratch_shapes=[
            pltpu.VMEM((P, 1), jnp.float32),
            pltpu.SMEM((1, 1), jnp.float32),
        ],
    )(xp, ytp)
    return out[0, 0]


# speedup vs baseline: 1.1614x; 1.1614x over previous
import jax
import jax.numpy as jnp
from jax.experimental import pallas as pl
from jax.experimental.pallas import tpu as pltpu

B = 8
P = 2048
Q = 2048
DPAD = 8
QT = 512
NQT = Q // QT
LANE = 128
NBLK = QT // LANE


def _chamfer_body(x_ref, yt_ref, out_ref, minx_ref, acc_ref):
    b = pl.program_id(0)
    qt = pl.program_id(1)

    x = x_ref[0]
    yt = yt_ref[0]

    x2 = jnp.sum(x * x, axis=1, keepdims=True)
    li = jax.lax.broadcasted_iota(jnp.int32, (P, DPAD), 1)
    aug_x = jnp.where(li == 3, x2, x)
    aug_x = jnp.where(li == 4, 1.0, aug_x)

    y2 = jnp.sum(yt * yt, axis=0, keepdims=True)
    si = jax.lax.broadcasted_iota(jnp.int32, (DPAD, QT), 0)
    aug_y = jnp.where(si == 3, 1.0, -2.0 * yt)
    aug_y = jnp.where(si == 4, y2, aug_y)

    d = jax.lax.dot_general(
        aug_x, aug_y, (((1,), (0,)), ((), ())),
        preferred_element_type=jnp.float32,
    )

    blks = [d[:, k * LANE:(k + 1) * LANE] for k in range(NBLK)]

    a = blks[0]
    for k in range(1, NBLK):
        a = jnp.minimum(a, blks[k])

    @pl.when(qt == 0)
    def _():
        minx_ref[...] = a

    @pl.when(qt != 0)
    def _():
        minx_ref[...] = jnp.minimum(minx_ref[...], a)

    s = 0.0
    for k in range(NBLK):
        my = jnp.min(blks[k], axis=0)
        s += jnp.sum(jnp.maximum(my, 0.0))

    @pl.when(jnp.logical_and(b == 0, qt == 0))
    def _():
        acc_ref[0, 0] = 0.0

    acc_ref[0, 0] += s

    @pl.when(qt == NQT - 1)
    def _():
        mx = jnp.min(minx_ref[...], axis=1)
        acc_ref[0, 0] += jnp.sum(jnp.maximum(mx, 0.0))

    @pl.when(jnp.logical_and(b == B - 1, qt == NQT - 1))
    def _():
        out_ref[0, 0] = acc_ref[0, 0] / (float(P) * float(B))


def kernel(x_hat, pos):
    xp = jnp.pad(x_hat, ((0, 0), (0, 0), (0, DPAD - 3)))
    ytp = jnp.pad(pos, ((0, 0), (0, 0), (0, DPAD - 3)))
    ytp = jnp.transpose(ytp, (0, 2, 1))

    out = pl.pallas_call(
        _chamfer_body,
        grid=(B, NQT),
        in_specs=[
            pl.BlockSpec((1, P, DPAD), lambda b, q: (b, 0, 0)),
            pl.BlockSpec((1, DPAD, QT), lambda b, q: (b, 0, q)),
        ],
        out_specs=pl.BlockSpec(
            (1, 1), lambda b, q: (0, 0), memory_space=pltpu.SMEM
        ),
        out_shape=jax.ShapeDtypeStruct((1, 1), jnp.float32),
        scratch_shapes=[
            pltpu.VMEM((P, LANE), jnp.float32),
            pltpu.SMEM((1, 1), jnp.float32),
        ],
    )(xp, ytp)
    return out[0, 0]


# speedup vs baseline: 1.7960x; 1.5464x over previous
import jax
import jax.numpy as jnp
from jax.experimental import pallas as pl
from jax.experimental.pallas import tpu as pltpu

B = 8
P = 2048
Q = 2048
DPAD = 8
LANE = 128
NBLK = Q // LANE


def _chamfer_body(x_ref, yt_ref, out_ref, acc_ref):
    b = pl.program_id(0)

    x = x_ref[0]
    yt = yt_ref[0]

    x2 = jnp.sum(x * x, axis=1, keepdims=True)
    li = jax.lax.broadcasted_iota(jnp.int32, (P, DPAD), 1)
    aug_x = x + jnp.where(li == 3, x2, 0.0) + (li == 4).astype(jnp.float32)

    y2 = jnp.sum(yt * yt, axis=0, keepdims=True)
    si = jax.lax.broadcasted_iota(jnp.int32, (DPAD, Q), 0)
    aug_y = jnp.where(si == 3, 1.0, -2.0 * yt)
    aug_y = jnp.where(si == 4, y2, aug_y)

    d = jax.lax.dot_general(
        aug_x, aug_y, (((1,), (0,)), ((), ())),
        preferred_element_type=jnp.float32,
    )

    s = 0.0
    a = None
    for k in range(NBLK):
        dblk = d[:, k * LANE:(k + 1) * LANE]
        a = dblk if a is None else jnp.minimum(a, dblk)
        my = jnp.min(dblk, axis=0)
        s += jnp.sum(jnp.maximum(my, 0.0))

    mx = jnp.min(a, axis=1)
    s += jnp.sum(jnp.maximum(mx, 0.0))

    @pl.when(b == 0)
    def _():
        acc_ref[0, 0] = 0.0

    acc_ref[0, 0] += s

    @pl.when(b == B - 1)
    def _():
        out_ref[0, 0] = acc_ref[0, 0] / (float(P) * float(B))


def kernel(x_hat, pos):
    xp = jnp.pad(x_hat, ((0, 0), (0, 0), (0, DPAD - 3)))
    ytp = jnp.pad(pos, ((0, 0), (0, 0), (0, DPAD - 3)))
    ytp = jnp.transpose(ytp, (0, 2, 1))

    out = pl.pallas_call(
        _chamfer_body,
        grid=(B,),
        in_specs=[
            pl.BlockSpec((1, P, DPAD), lambda b: (b, 0, 0)),
            pl.BlockSpec((1, DPAD, Q), lambda b: (b, 0, 0)),
        ],
        out_specs=pl.BlockSpec(
            (1, 1), lambda b: (0, 0), memory_space=pltpu.SMEM
        ),
        out_shape=jax.ShapeDtypeStruct((1, 1), jnp.float32),
        scratch_shapes=[
            pltpu.SMEM((1, 1), jnp.float32),
        ],
    )(xp, ytp)
    return out[0, 0]


# speedup vs baseline: 2.1531x; 1.1989x over previous
import jax
import jax.numpy as jnp
from jax.experimental import pallas as pl
from jax.experimental.pallas import tpu as pltpu

B = 8
P = 2048
Q = 2048
DPAD = 8
LANE = 128
NBLK = Q // LANE


def _chamfer_body(x_ref, yt_ref, out_ref, acc_ref):
    b = pl.program_id(0)

    x = x_ref[0]
    yt = yt_ref[0]

    x2 = jnp.sum(x * x, axis=1, keepdims=True)
    aug_x = jnp.concatenate(
        [x, x2, jnp.ones((P, 1), jnp.float32),
         jnp.zeros((P, DPAD - 5), jnp.float32)], axis=1)

    y2 = jnp.sum(yt * yt, axis=0, keepdims=True)
    aug_y = jnp.concatenate(
        [-2.0 * yt, jnp.ones((1, Q), jnp.float32), y2,
         jnp.zeros((DPAD - 5, Q), jnp.float32)], axis=0)

    d = jax.lax.dot_general(
        aug_x, aug_y, (((1,), (0,)), ((), ())),
        preferred_element_type=jnp.float32,
    )

    s = 0.0
    a = None
    for k in range(NBLK):
        dblk = d[:, k * LANE:(k + 1) * LANE]
        a = dblk if a is None else jnp.minimum(a, dblk)
        my = jnp.min(dblk, axis=0)
        s += jnp.sum(jnp.maximum(my, 0.0))

    mx = jnp.min(a, axis=1)
    s += jnp.sum(jnp.maximum(mx, 0.0))

    @pl.when(b == 0)
    def _():
        acc_ref[0, 0] = 0.0

    acc_ref[0, 0] += s

    @pl.when(b == B - 1)
    def _():
        out_ref[0, 0] = acc_ref[0, 0] / (float(P) * float(B))


def kernel(x_hat, pos):
    ytp = jnp.transpose(pos, (0, 2, 1))

    out = pl.pallas_call(
        _chamfer_body,
        grid=(B,),
        in_specs=[
            pl.BlockSpec((1, P, 3), lambda b: (b, 0, 0)),
            pl.BlockSpec((1, 3, Q), lambda b: (b, 0, 0)),
        ],
        out_specs=pl.BlockSpec(
            (1, 1), lambda b: (0, 0), memory_space=pltpu.SMEM
        ),
        out_shape=jax.ShapeDtypeStruct((1, 1), jnp.float32),
        scratch_shapes=[
            pltpu.SMEM((1, 1), jnp.float32),
        ],
    )(x_hat, ytp)
    return out[0, 0]
